# baseline (speedup 1.0000x reference)
import jax
import jax.numpy as jnp
from jax import lax
from jax.experimental import pallas as pl
from jax.experimental.pallas import tpu as pltpu
from jax.experimental.pallas import tpu_sc as plsc

BATCH = 1024
SEQ = 200
D = 128
NC = 2
NS = 16
NW = NC * NS
SEQ_PER_W = BATCH // NW
IDX_PER_W = SEQ_PER_W * SEQ
L0 = 104
L1 = SEQ - L0
NBLK = 2 * SEQ_PER_W
NBUF = 8
STAG = 4


def _body(x_hbm, tok_hbm, pos_hbm, out_hbm, idx_v, rows, psh, sems):
    sid = lax.axis_index("s")
    wid = sid * NC + lax.axis_index("c")
    ibase = pl.multiple_of(wid * IDX_PER_W, 8)
    pltpu.sync_copy(x_hbm.at[pl.ds(ibase, IDX_PER_W)], idx_v)
    @pl.when(sid == 0)
    def _stage():
        pltpu.sync_copy(pos_hbm.at[pl.ds(0, L0)], rows[0].at[pl.ds(0, L0)])
        pltpu.sync_copy(rows[0].at[pl.ds(0, L0)], psh.at[pl.ds(0, L0)])
        pltpu.sync_copy(pos_hbm.at[pl.ds(L0, L1)], rows[1].at[pl.ds(0, L1)])
        pltpu.sync_copy(rows[1].at[pl.ds(0, L1)], psh.at[pl.ds(L0, L1)])
    plsc.subcore_barrier()

    psem, gsem, ssem = sems

    def blk(k, b):
        ln = L0 if b % 2 == 0 else L1
        hoff = 0 if b % 2 == 0 else L0
        off = pl.multiple_of((k // 2) * SEQ + hoff, 8)
        return off, hoff, ln

    def gather_desc(k, b):
        off, _, ln = blk(k, b)
        return (tok_hbm.at[idx_v.at[pl.ds(off, ln)]],
                rows[b].at[pl.ds(0, ln)], gsem[b])

    def step(k, b):
        @pl.when(jnp.logical_and(k - STAG >= 0, k - STAG < NBLK))
        def _finish():
            j = k - STAG
            jb = (b + NBUF - STAG) % NBUF
            src, dst, sem = gather_desc(j, jb)
            pltpu.make_async_copy(src, dst, sem).wait()
            joff, _, jln = blk(j, jb)
            pltpu.async_copy(
                rows[jb].at[pl.ds(0, jln)],
                out_hbm.at[pl.ds(pl.multiple_of(wid * IDX_PER_W, 8) + joff,
                                 jln)],
                ssem[jb])

        @pl.when(k < NBLK)
        def _launch():
            _, hoff, ln = blk(k, b)
            @pl.when(k >= NBUF)
            def _reclaim():
                pltpu.make_async_copy(
                    rows[b].at[pl.ds(0, ln)], out_hbm.at[pl.ds(0, ln)],
                    ssem[b]).wait()
            pltpu.async_copy(psh.at[pl.ds(hoff, ln)],
                             rows[b].at[pl.ds(0, ln)], psem[b]).wait()
            src, dst, sem = gather_desc(k, b)
            pltpu.async_copy(src, dst, sem, add=True)

    @pl.loop(0, NBLK + STAG + NBUF - 1, step=NBUF)
    def _outer(g):
        for b in range(NBUF):
            step(g + b, b)

    for b in range(NBUF):
        ln = L0 if b % 2 == 0 else L1
        pltpu.make_async_copy(rows[b].at[pl.ds(0, ln)],
                              out_hbm.at[pl.ds(0, ln)], ssem[b]).wait()


def kernel(x, token_table, pos_table):
    mesh = plsc.VectorSubcoreMesh(core_axis_name="c", subcore_axis_name="s")
    f = pl.kernel(
        _body,
        out_type=jax.ShapeDtypeStruct((BATCH * SEQ, D), jnp.float32),
        mesh=mesh,
        scratch_types=[
            pltpu.VMEM((IDX_PER_W,), jnp.int32),
            [pltpu.VMEM((L0, D), jnp.float32) for _ in range(NBUF)],
            pltpu.VMEM_SHARED((SEQ, D), jnp.float32),
            [[pltpu.SemaphoreType.DMA for _ in range(NBUF)] for _ in range(3)],
        ],
    )
    out = f(x.reshape(-1), token_table, pos_table)
    return out.reshape(BATCH, SEQ, D)
